# Initial kernel scaffold; baseline (speedup 1.0000x reference)
import functools

import jax
import jax.numpy as jnp
from jax import lax
from jax.experimental import pallas as pl
from jax.experimental.pallas import tpu as pltpu
from jax.experimental.pallas import tpu_sc as plsc

BS, SEQ, IN_DIM, EMB = 4, 4096, 512, 1024
HID = EMB // 2
KK = max(1, int((SEQ - 2) * 0.3))
KPAD = ((KK + 7) // 8) * 8
ROWS = BS * KPAD
DUMP = ROWS
SC_OUT_ROWS = ROWS + 8
SEG = SEQ // 8
NGRP = SEG // 128
NCHUNK = 128 // 16


def _sc_body(text_hbm, feat_hbm, out_hbm, lens_hbm,
             tseg, srcidx, dstidx, rows, cnt, nzv, shared, sem):
    c = lax.axis_index("c")
    sub = lax.axis_index("s")
    b = 2 * c + sub // 8
    s = sub % 8
    seg_base = s * SEG

    pltpu.sync_copy(text_hbm.at[pl.ds(b * SEQ + seg_base, SEG)], tseg)

    zlane = jnp.zeros((16,), jnp.int32)
    for k in range(SEG // 16):
        t = tseg[pl.ds(k * 16, 16)]
        zlane = zlane + (t == 0).astype(jnp.int32)
    zseg = jnp.sum(zlane)
    nzv[...] = jnp.full((16,), zseg, jnp.int32)
    pltpu.sync_copy(nzv, shared.at[sub])
    plsc.subcore_barrier()
    pltpu.sync_copy(shared.at[pl.ds((sub // 8) * 8, 8)], cnt)

    nz_before = jnp.int32(0)
    nz_total = jnp.int32(0)
    for i in range(8):
        ci = cnt[i, 0]
        nz_before = nz_before + jnp.where(i < s, ci, 0)
        nz_total = nz_total + ci

    @pl.when(s == 0)
    def _():
        nzv[...] = jnp.full((16,), nz_total, jnp.int32)
        pltpu.sync_copy(nzv, lens_hbm.at[b])

    z_run = nz_before
    n_run = seg_base - nz_before
    for g in range(NGRP):
        any_valid = jnp.int32(0)
        for k in range(NCHUNK):
            off = g * 128 + k * 16
            t = tseg[pl.ds(off, 16)]
            zi = (t == 0).astype(jnp.int32)
            zc = plsc.cumsum(zi)
            nc = plsc.cumsum(1 - zi)
            rank = jnp.where(zi == 1,
                             z_run + zc - 1,
                             nz_total + n_run + nc - 1)
            valid = rank < KK
            lane = lax.iota(jnp.int32, 16)
            src = b * SEQ + seg_base + off + lane
            dst = jnp.where(valid, b * KPAD + rank, DUMP)
            srcidx[g, pl.ds(k * 16, 16)] = src
            dstidx[g, pl.ds(k * 16, 16)] = dst
            zcnt = jnp.sum(zi)
            z_run = z_run + zcnt
            n_run = n_run + (16 - zcnt)
            any_valid = any_valid + jnp.sum(valid.astype(jnp.int32))

        @pl.when(any_valid > 0)
        def _():
            pltpu.async_copy(feat_hbm.at[srcidx.at[g]], rows, sem).wait()
            pltpu.async_copy(rows, out_hbm.at[dstidx.at[g]], sem).wait()


@functools.partial(
    pl.kernel,
    mesh=plsc.VectorSubcoreMesh(core_axis_name="c", subcore_axis_name="s"),
    out_type=[
        jax.ShapeDtypeStruct((SC_OUT_ROWS, IN_DIM), jnp.float32),
        jax.ShapeDtypeStruct((BS, 16), jnp.int32),
    ],
    scratch_types=[
        pltpu.VMEM((SEG,), jnp.int32),
        pltpu.VMEM((NGRP, 128), jnp.int32),
        pltpu.VMEM((NGRP, 128), jnp.int32),
        pltpu.VMEM((128, IN_DIM), jnp.float32),
        pltpu.VMEM((8, 16), jnp.int32),
        pltpu.VMEM((16,), jnp.int32),
        pltpu.VMEM_SHARED((16, 16), jnp.int32),
        pltpu.SemaphoreType.DMA,
    ],
)
def _sc_gather(text_hbm, feat_hbm, out_hbm, lens_hbm, *scratch):
    _sc_body(text_hbm, feat_hbm, out_hbm, lens_hbm, *scratch)


def _tc_body(x_ref, wlt_ref, w0t_ref, w1t_ref, blin_ref, b0_ref, b1_ref,
             g0_ref, be0_ref, plens_ref, out_ref):
    ridx = lax.broadcasted_iota(jnp.int32, (SC_OUT_ROWS, 1), 0)
    row_ok = (ridx % KPAD < KK) & (ridx < ROWS)

    x = jnp.where(row_ok, x_ref[...], 0.0)
    ssq = jnp.sum(x * x, axis=1, keepdims=True)
    xn = x / jnp.maximum(jnp.sqrt(ssq), 1e-6)

    h = jnp.dot(xn, w0t_ref[...], preferred_element_type=jnp.float32) + b0_ref[...]
    denom = jnp.float32(BS * KK)
    mu = jnp.sum(jnp.where(row_ok, h, 0.0), axis=0, keepdims=True) / denom
    d = h - mu
    var = jnp.sum(jnp.where(row_ok, d * d, 0.0), axis=0, keepdims=True) / denom
    hn = d / jnp.sqrt(var + 1e-5) * g0_ref[...] + be0_ref[...]
    r = jnp.maximum(hn, 0.0)

    fused = (jnp.dot(r, w1t_ref[...], preferred_element_type=jnp.float32)
             + jnp.dot(xn, wlt_ref[...], preferred_element_type=jnp.float32)
             + b1_ref[...] + blin_ref[...])

    neg = jnp.float32(-jnp.inf)
    for b in range(BS):
        seg = fused[b * KPAD:(b + 1) * KPAD, :]
        pm = lax.broadcasted_iota(jnp.int32, (KPAD, 1), 0) < plens_ref[b]
        out_ref[b, :] = jnp.max(jnp.where(pm, seg, neg), axis=0)


def _tc_dense(x, wlt, w0t, w1t, blin, b0, b1, g0, be0, plens):
    vspec = pl.BlockSpec(memory_space=pltpu.VMEM)
    return pl.pallas_call(
        _tc_body,
        out_shape=jax.ShapeDtypeStruct((BS, EMB), jnp.float32),
        in_specs=[vspec] * 9 + [pl.BlockSpec(memory_space=pltpu.SMEM)],
        out_specs=vspec,
        compiler_params=pltpu.CompilerParams(
            vmem_limit_bytes=128 * 1024 * 1024),
    )(x, wlt, w0t, w1t, blin, b0, b1, g0, be0, plens)


def kernel(features, text, atten, W_lin, b_lin, W0, b0, g0, be0, W1, b1):
    del atten
    feat_flat = features.reshape(BS * SEQ, IN_DIM)
    text_flat = text.reshape(BS * SEQ).astype(jnp.int32)

    feats_sc, lens = _sc_gather(text_flat, feat_flat)

    token_lens = SEQ - lens[:, 0]
    plens = jnp.clip(token_lens - 2, 1, KK).astype(jnp.int32)

    out = _tc_dense(
        feats_sc,
        W_lin.T, W0.T, W1.T,
        b_lin.reshape(1, EMB), b0.reshape(1, HID), b1.reshape(1, EMB),
        g0.reshape(1, HID), be0.reshape(1, HID),
        plens,
    )
    return out.astype(jnp.float32)


# speedup vs baseline: 68.5590x; 68.5590x over previous
import functools

import jax
import jax.numpy as jnp
from jax import lax
from jax.experimental import pallas as pl
from jax.experimental.pallas import tpu as pltpu
from jax.experimental.pallas import tpu_sc as plsc

BS, SEQ, IN_DIM, EMB = 4, 4096, 512, 1024
HID = EMB // 2
KK = max(1, int((SEQ - 2) * 0.3))
KPAD = ((KK + 7) // 8) * 8
ROWS = BS * KPAD
SC_OUT_ROWS = ROWS + 32
SEG = SEQ // 8
NGRP = SEG // 128
NCHUNK = 128 // 16


def _sc_body(text_hbm, feat_hbm, out_hbm, lens_hbm,
             trow, srcidx, dstidx, rows, nzv, sem):
    c = lax.axis_index("c")
    sub = lax.axis_index("s")
    b = 2 * c + sub // 8
    s = sub % 8
    seg_base = s * SEG
    wid = 16 * c + sub
    dump = ROWS + wid

    pltpu.sync_copy(text_hbm.at[pl.ds(b * SEQ, SEQ)], trow)

    z_before = jnp.zeros((16,), jnp.int32)
    z_total = jnp.zeros((16,), jnp.int32)
    for j in range(8):
        zseg = jnp.zeros((16,), jnp.int32)
        for k in range(SEG // 16):
            t = trow[pl.ds(j * SEG + k * 16, 16)]
            zseg = zseg + plsc.all_reduce_population_count(t == 0)
        z_total = z_total + zseg
        z_before = z_before + zseg * jnp.int32(j < s)

    @pl.when(s == 0)
    def _():
        nzv[...] = z_total
        pltpu.sync_copy(nzv, lens_hbm.at[b])

    lane = lax.iota(jnp.int32, 16)
    z_run = z_before
    n_run = jnp.full((16,), seg_base, jnp.int32) - z_before
    for g in range(NGRP):
        n_valid = jnp.zeros((16,), jnp.int32)
        for k in range(NCHUNK):
            off = seg_base + g * 128 + k * 16
            t = trow[pl.ds(off, 16)]
            zb = t == 0
            zi = zb.astype(jnp.int32)
            zc = plsc.cumsum(zi)
            nc = lane + 1 - zc
            rank = jnp.where(zb,
                             z_run + zc - 1,
                             z_total + n_run + nc - 1)
            valid = rank < KK
            src = b * SEQ + off + lane
            dst = jnp.where(valid, b * KPAD + rank, dump)
            srcidx[g, pl.ds(k * 16, 16)] = src
            dstidx[g, pl.ds(k * 16, 16)] = dst
            zcnt = plsc.all_reduce_population_count(zb)
            z_run = z_run + zcnt
            n_run = n_run + (16 - zcnt)
            n_valid = n_valid + plsc.all_reduce_population_count(valid)

        nv = n_valid[0]

        @pl.when(nv > 0)
        def _():
            pltpu.async_copy(feat_hbm.at[srcidx.at[g]], rows, sem).wait()
            pltpu.async_copy(rows, out_hbm.at[dstidx.at[g]], sem).wait()


@functools.partial(
    pl.kernel,
    mesh=plsc.VectorSubcoreMesh(core_axis_name="c", subcore_axis_name="s"),
    compiler_params=pltpu.CompilerParams(needs_layout_passes=False),
    out_type=[
        jax.ShapeDtypeStruct((SC_OUT_ROWS, IN_DIM), jnp.float32),
        jax.ShapeDtypeStruct((BS, 16), jnp.int32),
    ],
    scratch_types=[
        pltpu.VMEM((SEQ,), jnp.int32),
        pltpu.VMEM((NGRP, 128), jnp.int32),
        pltpu.VMEM((NGRP, 128), jnp.int32),
        pltpu.VMEM((128, IN_DIM), jnp.float32),
        pltpu.VMEM((16,), jnp.int32),
        pltpu.SemaphoreType.DMA,
    ],
)
def _sc_gather(text_hbm, feat_hbm, out_hbm, lens_hbm, *scratch):
    _sc_body(text_hbm, feat_hbm, out_hbm, lens_hbm, *scratch)


def _tc_body(x_ref, wlt_ref, w0t_ref, w1t_ref, blin_ref, b0_ref, b1_ref,
             g0_ref, be0_ref, plens_ref, out_ref):
    ridx = lax.broadcasted_iota(jnp.int32, (SC_OUT_ROWS, 1), 0)
    row_ok = (ridx % KPAD < KK) & (ridx < ROWS)

    x = jnp.where(row_ok, x_ref[...], 0.0)
    ssq = jnp.sum(x * x, axis=1, keepdims=True)
    xn = x / jnp.maximum(jnp.sqrt(ssq), 1e-6)

    h = jnp.dot(xn, w0t_ref[...], preferred_element_type=jnp.float32) + b0_ref[...]
    denom = jnp.float32(BS * KK)
    mu = jnp.sum(jnp.where(row_ok, h, 0.0), axis=0, keepdims=True) / denom
    d = h - mu
    var = jnp.sum(jnp.where(row_ok, d * d, 0.0), axis=0, keepdims=True) / denom
    hn = d / jnp.sqrt(var + 1e-5) * g0_ref[...] + be0_ref[...]
    r = jnp.maximum(hn, 0.0)

    fused = (jnp.dot(r, w1t_ref[...], preferred_element_type=jnp.float32)
             + jnp.dot(xn, wlt_ref[...], preferred_element_type=jnp.float32)
             + b1_ref[...] + blin_ref[...])

    neg = jnp.float32(-jnp.inf)
    for b in range(BS):
        seg = fused[b * KPAD:(b + 1) * KPAD, :]
        pm = lax.broadcasted_iota(jnp.int32, (KPAD, 1), 0) < plens_ref[b]
        out_ref[b, :] = jnp.max(jnp.where(pm, seg, neg), axis=0)


def _tc_dense(x, wlt, w0t, w1t, blin, b0, b1, g0, be0, plens):
    vspec = pl.BlockSpec(memory_space=pltpu.VMEM)
    return pl.pallas_call(
        _tc_body,
        out_shape=jax.ShapeDtypeStruct((BS, EMB), jnp.float32),
        in_specs=[vspec] * 9 + [pl.BlockSpec(memory_space=pltpu.SMEM)],
        out_specs=vspec,
        compiler_params=pltpu.CompilerParams(
            vmem_limit_bytes=128 * 1024 * 1024),
    )(x, wlt, w0t, w1t, blin, b0, b1, g0, be0, plens)


def kernel(features, text, atten, W_lin, b_lin, W0, b0, g0, be0, W1, b1):
    del atten
    feat_flat = features.reshape(BS * SEQ, IN_DIM)
    text_flat = text.reshape(BS * SEQ).astype(jnp.int32)

    feats_sc, lens = _sc_gather(text_flat, feat_flat)

    token_lens = SEQ - lens[:, 0]
    plens = jnp.clip(token_lens - 2, 1, KK).astype(jnp.int32)

    out = _tc_dense(
        feats_sc,
        W_lin.T, W0.T, W1.T,
        b_lin.reshape(1, EMB), b0.reshape(1, HID), b1.reshape(1, EMB),
        g0.reshape(1, HID), be0.reshape(1, HID),
        plens,
    )
    return out.astype(jnp.float32)


# speedup vs baseline: 82.0049x; 1.1961x over previous
import functools

import jax
import jax.numpy as jnp
from jax import lax
from jax.experimental import pallas as pl
from jax.experimental.pallas import tpu as pltpu
from jax.experimental.pallas import tpu_sc as plsc

BS, SEQ, IN_DIM, EMB = 4, 4096, 512, 1024
HID = EMB // 2
KK = max(1, int((SEQ - 2) * 0.3))
KPAD = 1280
PT = KPAD // 8
ROWS = BS * KPAD
NCH = SEQ // 16


def _sc_body(text_hbm, feat_hbm, out_hbm, lens_hbm,
             trow, idxa, idxb, rowsa, rowsb, nzv, sema, semb):
    c = lax.axis_index("c")
    sub = lax.axis_index("s")
    wid = 16 * c + sub
    b = wid // 8
    s = wid % 8
    lo = s * PT
    dstbase = b * KPAD + lo

    pltpu.sync_copy(text_hbm.at[pl.ds(b * SEQ, SEQ)], trow)

    lane = lax.iota(jnp.int32, 16)

    def abody(i, zt):
        t = trow[pl.ds(i * 16, 16)]
        return zt + plsc.all_reduce_population_count(t == 0)
    z_total = lax.fori_loop(0, NCH, abody, jnp.zeros((16,), jnp.int32))

    @pl.when(s == 0)
    def _():
        nzv[...] = z_total
        pltpu.sync_copy(nzv, lens_hbm.at[b])

    lo_vec = jnp.full((16,), lo, jnp.int32)
    src_base = jnp.full((16,), b * SEQ, jnp.int32) + lane

    def bbody(i, z_run):
        t = trow[pl.ds(i * 16, 16)]
        zb = t == 0
        zc = plsc.cumsum(zb.astype(jnp.int32))
        pos = i * 16 + lane
        rank = jnp.where(zb, z_run + zc,
                         z_total + pos + 1 - z_run - zc) - 1
        slot = rank - lo_vec
        ma = (slot >= 0) & (slot < 128)
        mb = (slot >= 128) & (slot < PT)
        srcv = src_base + i * 16
        plsc.store_scatter(idxa, [jnp.clip(slot, 0, 127)], srcv, mask=ma)
        plsc.store_scatter(idxb, [jnp.clip(slot - 128, 0, PT - 129)],
                           srcv, mask=mb)
        return z_run + plsc.all_reduce_population_count(zb)
    lax.fori_loop(0, NCH, bbody, jnp.zeros((16,), jnp.int32))

    ca = pltpu.async_copy(feat_hbm.at[idxa], rowsa, sema)
    cb = pltpu.async_copy(feat_hbm.at[idxb], rowsb, semb)
    ca.wait()
    pltpu.sync_copy(rowsa, out_hbm.at[pl.ds(dstbase, 128)])
    cb.wait()
    pltpu.sync_copy(rowsb, out_hbm.at[pl.ds(dstbase + 128, PT - 128)])


@functools.partial(
    pl.kernel,
    mesh=plsc.VectorSubcoreMesh(core_axis_name="c", subcore_axis_name="s"),
    compiler_params=pltpu.CompilerParams(needs_layout_passes=False),
    out_type=[
        jax.ShapeDtypeStruct((ROWS, IN_DIM), jnp.float32),
        jax.ShapeDtypeStruct((BS, 16), jnp.int32),
    ],
    scratch_types=[
        pltpu.VMEM((SEQ,), jnp.int32),
        pltpu.VMEM((128,), jnp.int32),
        pltpu.VMEM((PT - 128,), jnp.int32),
        pltpu.VMEM((128, IN_DIM), jnp.float32),
        pltpu.VMEM((PT - 128, IN_DIM), jnp.float32),
        pltpu.VMEM((16,), jnp.int32),
        pltpu.SemaphoreType.DMA,
        pltpu.SemaphoreType.DMA,
    ],
)
def _sc_gather(text_hbm, feat_hbm, out_hbm, lens_hbm, *scratch):
    _sc_body(text_hbm, feat_hbm, out_hbm, lens_hbm, *scratch)


def _tc_body(x_ref, wlt_ref, w0t_ref, w1t_ref, blin_ref, b0_ref, b1_ref,
             g0_ref, be0_ref, plens_ref, out_ref):
    ridx = lax.broadcasted_iota(jnp.int32, (ROWS, 1), 0)
    row_ok = ridx % KPAD < KK

    x = jnp.where(row_ok, x_ref[...], 0.0)
    ssq = jnp.sum(x * x, axis=1, keepdims=True)
    xn = x / jnp.maximum(jnp.sqrt(ssq), 1e-6)
    xb = xn.astype(jnp.bfloat16)

    h = jnp.dot(xb, w0t_ref[...].astype(jnp.bfloat16),
                preferred_element_type=jnp.float32) + b0_ref[...]
    denom = jnp.float32(BS * KK)
    mu = jnp.sum(jnp.where(row_ok, h, 0.0), axis=0, keepdims=True) / denom
    d = h - mu
    var = jnp.sum(jnp.where(row_ok, d * d, 0.0), axis=0, keepdims=True) / denom
    hn = d / jnp.sqrt(var + 1e-5) * g0_ref[...] + be0_ref[...]
    r = jnp.maximum(hn, 0.0)

    fused = (jnp.dot(r.astype(jnp.bfloat16), w1t_ref[...].astype(jnp.bfloat16),
                     preferred_element_type=jnp.float32)
             + jnp.dot(xb, wlt_ref[...].astype(jnp.bfloat16),
                       preferred_element_type=jnp.float32)
             + b1_ref[...] + blin_ref[...])

    neg = jnp.float32(-jnp.inf)
    for b in range(BS):
        seg = fused[b * KPAD:(b + 1) * KPAD, :]
        pm = lax.broadcasted_iota(jnp.int32, (KPAD, 1), 0) < plens_ref[b]
        out_ref[b, :] = jnp.max(jnp.where(pm, seg, neg), axis=0)


def _tc_dense(x, wlt, w0t, w1t, blin, b0, b1, g0, be0, plens):
    vspec = pl.BlockSpec(memory_space=pltpu.VMEM)
    return pl.pallas_call(
        _tc_body,
        out_shape=jax.ShapeDtypeStruct((BS, EMB), jnp.float32),
        in_specs=[vspec] * 9 + [pl.BlockSpec(memory_space=pltpu.SMEM)],
        out_specs=vspec,
        compiler_params=pltpu.CompilerParams(
            vmem_limit_bytes=128 * 1024 * 1024),
    )(x, wlt, w0t, w1t, blin, b0, b1, g0, be0, plens)


def kernel(features, text, atten, W_lin, b_lin, W0, b0, g0, be0, W1, b1):
    del atten
    feat_flat = features.reshape(BS * SEQ, IN_DIM)
    text_flat = text.reshape(BS * SEQ).astype(jnp.int32)

    feats_sc, lens = _sc_gather(text_flat, feat_flat)

    token_lens = SEQ - lens[:, 0]
    plens = jnp.clip(token_lens - 2, 1, KK).astype(jnp.int32)

    out = _tc_dense(
        feats_sc,
        W_lin.T, W0.T, W1.T,
        b_lin.reshape(1, EMB), b0.reshape(1, HID), b1.reshape(1, EMB),
        g0.reshape(1, HID), be0.reshape(1, HID),
        plens,
    )
    return out.astype(jnp.float32)
